# baseline (speedup 1.0000x reference)
import jax
import jax.numpy as jnp
from jax.experimental import pallas as pl

TOTAL_TOKENS = 32768
D = 1024
BLOCK_ROWS = 512


def _copy_body(x_ref, o_ref):
    o_ref[...] = x_ref[...]


def kernel(flat, row_splits):
    values = pl.pallas_call(
        _copy_body,
        grid=(TOTAL_TOKENS // BLOCK_ROWS,),
        in_specs=[pl.BlockSpec((BLOCK_ROWS, D), lambda i: (i, 0))],
        out_specs=pl.BlockSpec((BLOCK_ROWS, D), lambda i: (i, 0)),
        out_shape=jax.ShapeDtypeStruct((TOTAL_TOKENS, D), jnp.float32),
    )(flat)
    return (values, row_splits)
